# baseline (speedup 1.0000x reference)
import functools

import jax
import jax.numpy as jnp
from jax.experimental import pallas as pl

_INPUT_DIM = 4096
_NUM_TOWERS = 64
_TOP_K = 8
_BLOCK_ROWS = 1024


def _router_body(x_ref, w_ref, b_ref, sel_ref, scores_ref, idx_ref):
    logits = jnp.dot(x_ref[...], w_ref[...], preferred_element_type=jnp.float32)
    logits = logits + b_ref[...]
    vals = logits
    ms = []
    idxf = None
    for j in range(_TOP_K):
        m = jnp.max(vals, axis=1, keepdims=True)
        c = vals == m
        vals = jnp.where(c, -jnp.inf, vals)
        ms.append(m)
        p = jnp.dot(c.astype(jnp.float32),
                    sel_ref[pl.ds(j * _NUM_TOWERS, _NUM_TOWERS), :],
                    preferred_element_type=jnp.float32)
        idxf = p if idxf is None else idxf + p
    top = jnp.concatenate(ms, axis=1)
    e = jnp.exp(top - top[:, :1])
    scores_ref[...] = e / jnp.sum(e, axis=1, keepdims=True)
    idx_ref[...] = idxf.astype(jnp.int32)


@functools.partial(jax.jit, static_argnames=("interpret",))
def kernel(x, gate_weight, gate_bias, interpret=False):
    n_tokens = x.shape[0]
    wt = gate_weight.T
    b = gate_bias.reshape(1, _NUM_TOWERS)
    colv = jnp.arange(_NUM_TOWERS, dtype=jnp.float32)
    sel = jnp.zeros((_TOP_K * _NUM_TOWERS, _TOP_K), jnp.float32)
    for j in range(_TOP_K):
        sel = sel.at[j * _NUM_TOWERS:(j + 1) * _NUM_TOWERS, j].set(colv)
    grid = (n_tokens // _BLOCK_ROWS,)
    scores, idx = pl.pallas_call(
        _router_body,
        grid=grid,
        in_specs=[
            pl.BlockSpec((_BLOCK_ROWS, _INPUT_DIM), lambda i: (i, 0)),
            pl.BlockSpec((_INPUT_DIM, _NUM_TOWERS), lambda i: (0, 0)),
            pl.BlockSpec((1, _NUM_TOWERS), lambda i: (0, 0)),
            pl.BlockSpec((_TOP_K * _NUM_TOWERS, _TOP_K), lambda i: (0, 0)),
        ],
        out_specs=[
            pl.BlockSpec((_BLOCK_ROWS, _TOP_K), lambda i: (i, 0)),
            pl.BlockSpec((_BLOCK_ROWS, _TOP_K), lambda i: (i, 0)),
        ],
        out_shape=[
            jax.ShapeDtypeStruct((n_tokens, _TOP_K), jnp.float32),
            jax.ShapeDtypeStruct((n_tokens, _TOP_K), jnp.int32),
        ],
        interpret=interpret,
    )(x, wt, b, sel)
    return scores, idx
